# Initial kernel scaffold; baseline (speedup 1.0000x reference)
import jax
import jax.numpy as jnp
from jax.experimental import pallas as pl


def kernel(indices_words, table):
    raise NotImplementedError("write your pallas kernel here")


# speedup vs baseline: 1.6835x; 1.6835x over previous
import functools

import jax
import jax.numpy as jnp
from jax import lax
from jax.experimental import pallas as pl
from jax.experimental.pallas import tpu as pltpu
from jax.experimental.pallas import tpu_sc as plsc

EMBEDDING_SIZE = 64
BATCH = 16384
HIST_LEN = 50

_B_TOTAL = BATCH * HIST_LEN
_CHUNK = 128
_N_BLOCKS = _B_TOTAL // _CHUNK

_info = plsc.get_sparse_core_info()
_NC, _NS = _info.num_cores, _info.num_subcores
_NW = _NC * _NS
_BLKS_W = _N_BLOCKS // _NW


def _make_gather():
    mesh = plsc.VectorSubcoreMesh(core_axis_name="c", subcore_axis_name="s")

    @functools.partial(
        pl.kernel,
        mesh=mesh,
        compiler_params=pltpu.CompilerParams(use_tc_tiling_on_sc=False),
        out_type=jax.ShapeDtypeStruct((_B_TOTAL, EMBEDDING_SIZE), jnp.float32),
        scratch_types=[
            pltpu.VMEM((_BLKS_W, _CHUNK), jnp.int32),
            pltpu.VMEM((_CHUNK, EMBEDDING_SIZE), jnp.float32),
            pltpu.SemaphoreType.DMA,
        ],
    )
    def gather_kernel(idx_hbm, table_hbm, out_hbm, idx_v, rows_v, sem):
        wid = lax.axis_index("s") * _NC + lax.axis_index("c")
        blk0 = wid * _BLKS_W
        pltpu.sync_copy(idx_hbm.at[pl.ds(blk0, _BLKS_W)], idx_v)

        def chunk(j, carry):
            pltpu.async_copy(table_hbm.at[idx_v.at[j]], rows_v, sem).wait()
            pltpu.sync_copy(
                rows_v, out_hbm.at[pl.ds((blk0 + j) * _CHUNK, _CHUNK)]
            )
            return carry

        lax.fori_loop(0, _BLKS_W, chunk, 0)

    return gather_kernel


_gather = _make_gather()


def kernel(indices_words, table):
    idx2d = indices_words.astype(jnp.int32).reshape(_N_BLOCKS, _CHUNK)
    flat = _gather(idx2d, table)
    return flat.reshape(BATCH, HIST_LEN, EMBEDDING_SIZE)


# speedup vs baseline: 1.8723x; 1.1122x over previous
import functools

import jax
import jax.numpy as jnp
from jax import lax
from jax.experimental import pallas as pl
from jax.experimental.pallas import tpu as pltpu
from jax.experimental.pallas import tpu_sc as plsc

EMBEDDING_SIZE = 64
BATCH = 16384
HIST_LEN = 50

_B_TOTAL = BATCH * HIST_LEN
_CHUNK = 128
_N_BLOCKS = _B_TOTAL // _CHUNK

_info = plsc.get_sparse_core_info()
_NC, _NS = _info.num_cores, _info.num_subcores
_NW = _NC * _NS
_BLKS_W = _N_BLOCKS // _NW

_K = 4
_GROUP = _K * _CHUNK
_NG = _BLKS_W // _K


def _make_gather():
    mesh = plsc.VectorSubcoreMesh(core_axis_name="c", subcore_axis_name="s")

    @functools.partial(
        pl.kernel,
        mesh=mesh,
        compiler_params=pltpu.CompilerParams(use_tc_tiling_on_sc=False),
        out_type=jax.ShapeDtypeStruct((_B_TOTAL, EMBEDDING_SIZE), jnp.float32),
        scratch_types=[
            pltpu.VMEM((_BLKS_W, _CHUNK), jnp.int32),
            pltpu.VMEM((_GROUP, EMBEDDING_SIZE), jnp.float32),
            pltpu.VMEM((_GROUP, EMBEDDING_SIZE), jnp.float32),
            pltpu.SemaphoreType.DMA,
            pltpu.SemaphoreType.DMA,
        ],
    )
    def gather_kernel(idx_hbm, table_hbm, out_hbm, idx_v, rows0, rows1,
                      gsem, ssem):
        wid = lax.axis_index("s") * _NC + lax.axis_index("c")
        blk0 = wid * _BLKS_W
        pltpu.sync_copy(idx_hbm.at[pl.ds(blk0, _BLKS_W)], idx_v)

        rows = (rows0, rows1)

        def do_group(g, set_i, drain_prev):
            buf = rows[set_i]
            descs = [
                pltpu.async_copy(
                    table_hbm.at[idx_v.at[g * _K + b]],
                    buf.at[pl.ds(b * _CHUNK, _CHUNK)],
                    gsem,
                )
                for b in range(_K)
            ]
            for d in descs:
                d.wait()
            if drain_prev:
                pltpu.make_async_copy(
                    rows[1 - set_i], out_hbm.at[pl.ds(0, _GROUP)], ssem
                ).wait()
            pltpu.async_copy(
                buf,
                out_hbm.at[pl.ds((blk0 + g * _K) * _CHUNK, _GROUP)],
                ssem,
            )

        do_group(0, 0, False)
        do_group(1, 1, True)

        def body(t, carry):
            do_group(2 * t + 2, 0, True)
            do_group(2 * t + 3, 1, True)
            return carry

        lax.fori_loop(0, (_NG - 2) // 2, body, 0)
        pltpu.make_async_copy(
            rows1, out_hbm.at[pl.ds(0, _GROUP)], ssem
        ).wait()

    return gather_kernel


_gather = _make_gather()


def kernel(indices_words, table):
    idx2d = indices_words.astype(jnp.int32).reshape(_N_BLOCKS, _CHUNK)
    flat = _gather(idx2d, table)
    return flat.reshape(BATCH, HIST_LEN, EMBEDDING_SIZE)
